# Initial kernel scaffold; baseline (speedup 1.0000x reference)
import jax
import jax.numpy as jnp
from jax.experimental import pallas as pl


def kernel(x, W_ih, W_hh, b_ih, b_hh):
    raise NotImplementedError("write your pallas kernel here")


# speedup vs baseline: 17.9517x; 17.9517x over previous
import jax
import jax.numpy as jnp
from jax.experimental import pallas as pl
from jax.experimental.pallas import tpu as pltpu

_TT = 512
_U = 8


def _sigmoid(x):
    return 0.5 * jnp.tanh(0.5 * x) + 0.5


def kernel(x, W_ih, W_hh, b_ih, b_hh):
    B, T, IN = x.shape
    H = W_hh.shape[1]
    G4 = 4 * H
    NT = T // _TT
    NG = _TT // _U

    Wt = W_hh.T
    misc = jnp.stack([W_ih[:, 0], b_ih + b_hh])
    xg = jnp.transpose(x[:, :, 0].reshape(B, T // _U, _U), (1, 0, 2))

    def body(x_ref, w_ref, misc_ref, hist_ref, sfin_ref, h_s, c_s):
        j = pl.program_id(0)

        @pl.when(j == 0)
        def _():
            h_s[...] = jnp.zeros_like(h_s)
            c_s[...] = jnp.zeros_like(c_s)

        W = w_ref[...]
        wr = misc_ref[0:1, :]
        bs = misc_ref[1:2, :]

        def group(gi, carry):
            h, c = carry
            t8 = gi * _U
            xgrp = x_ref[pl.ds(gi, 1), :, :].reshape(B, _U)
            for u in range(_U):
                gx = xgrp[:, u : u + 1] * wr + bs
                gates = jnp.dot(h, W, preferred_element_type=jnp.float32) + gx
                i_g = _sigmoid(gates[:, :H])
                f_g = _sigmoid(gates[:, H : 2 * H])
                g_g = jnp.tanh(gates[:, 2 * H : 3 * H])
                o_g = _sigmoid(gates[:, 3 * H :])
                c_new = f_g * c + i_g * g_g
                h_new = o_g * jnp.tanh(c_new)
                h = 0.5 * (h + h_new)
                c = 0.5 * (c + c_new)
                hist_ref[pl.ds(t8 + u, 1), :, :] = h[None]
            return (h, c)

        h, c = jax.lax.fori_loop(0, NG, group, (h_s[...], c_s[...]))
        h_s[...] = h
        c_s[...] = c

        @pl.when(j == NT - 1)
        def _():
            sfin_ref[:, :H] = h
            sfin_ref[:, H:] = c

    hist, sfin = pl.pallas_call(
        body,
        grid=(NT,),
        in_specs=[
            pl.BlockSpec((_TT // _U, B, _U), lambda j: (j, 0, 0)),
            pl.BlockSpec((H, G4), lambda j: (0, 0)),
            pl.BlockSpec((2, G4), lambda j: (0, 0)),
        ],
        out_specs=[
            pl.BlockSpec((_TT, B, H), lambda j: (j, 0, 0)),
            pl.BlockSpec((B, 2 * H), lambda j: (0, 0)),
        ],
        out_shape=[
            jax.ShapeDtypeStruct((T, B, H), jnp.float32),
            jax.ShapeDtypeStruct((B, 2 * H), jnp.float32),
        ],
        scratch_shapes=[
            pltpu.VMEM((B, H), jnp.float32),
            pltpu.VMEM((B, H), jnp.float32),
        ],
        compiler_params=pltpu.CompilerParams(
            dimension_semantics=("arbitrary",),
        ),
    )(xg, Wt, misc)

    return jnp.swapaxes(hist, 0, 1), sfin
